# Initial kernel scaffold; baseline (speedup 1.0000x reference)
import functools

import jax
import jax.numpy as jnp
from jax.experimental import pallas as pl
from jax.experimental.pallas import tpu as pltpu

_B = 1000
_NEG_INF = jnp.float32(float("-inf"))


def _seg_kernel(lev_ref, x_ref, Wproj_ref, W1a_ref, bproj_ref,
                sums_ref, cntrow_ref, cntb_ref, Wc_ref, bc_ref, *, nlev):
    i = pl.program_id(0)
    lev = lev_ref[0]
    B = lev.shape[0]
    x = x_ref[...]
    onehot = (lev == jax.lax.broadcasted_iota(jnp.int32, (B, nlev), 1)
              ).astype(jnp.float32)
    psums = jax.lax.dot_general(onehot, x, (((0,), (0,)), ((), ())),
                                preferred_element_type=jnp.float32)
    pcnt_row = jnp.sum(onehot, axis=0, keepdims=True)
    pcnt_b = jax.lax.dot_general(onehot, jnp.ones_like(x),
                                 (((0,), (0,)), ((), ())),
                                 preferred_element_type=jnp.float32)

    @pl.when(i == 0)
    def _():
        sums_ref[...] = psums
        cntrow_ref[...] = pcnt_row
        cntb_ref[...] = pcnt_b
        for l in range(nlev):
            Wc_ref[l] = jax.lax.dot_general(
                Wproj_ref[l], W1a_ref[...], (((1,), (0,)), ((), ())),
                preferred_element_type=jnp.float32)
        bc_ref[...] = jax.lax.dot_general(
            bproj_ref[...], W1a_ref[...], (((1,), (0,)), ((), ())),
            preferred_element_type=jnp.float32)

    @pl.when(i > 0)
    def _():
        sums_ref[...] += psums
        cntrow_ref[...] += pcnt_row
        cntb_ref[...] += pcnt_b


def _attn_kernel(lev_ref, x_ref, sums_ref, cntrow_ref, cntb_ref, Wc_ref,
                 bc_ref, Wproj_ref, bproj_ref, W1b_ref, b1_ref, W2t_ref,
                 b2_ref, Wo_ref, bo_ref, gamma_ref, beta_ref,
                 out_ref, ph_ref, *, nlev):
    lev = lev_ref[0]
    B = lev.shape[0]
    x = x_ref[...]

    cnt_b = cntb_ref[...]
    means_x = sums_ref[...] / jnp.maximum(cnt_b, 1.0)
    rows = [jax.lax.dot_general(means_x[l:l + 1], Wproj_ref[l],
                                (((1,), (0,)), ((), ())),
                                preferred_element_type=jnp.float32)
            for l in range(nlev)]
    means_h = jnp.concatenate(rows, axis=0) + bproj_ref[...]
    means_h = jnp.where(cnt_b > 0.0, means_h, 0.0)
    part_m = jax.lax.dot_general(means_h, W1b_ref[...],
                                 (((1,), (0,)), ((), ())),
                                 preferred_element_type=jnp.float32)
    part_m = part_m + b1_ref[...]

    ph_ref[...] = jnp.zeros_like(ph_ref)
    for l in range(nlev):
        m = lev == l

        @pl.when(jnp.any(m))
        def _(l=l, m=m):
            ph_ref[...] += jax.lax.dot_general(
                jnp.where(m, x, 0.0), Wc_ref[l], (((1,), (0,)), ((), ())),
                preferred_element_type=jnp.float32)

    onehot = (lev == jax.lax.broadcasted_iota(jnp.int32, (B, nlev), 1)
              ).astype(jnp.float32)
    part_h = ph_ref[...] + jax.lax.dot_general(
        onehot, bc_ref[...], (((1,), (0,)), ((), ())),
        preferred_element_type=jnp.float32)

    w2 = W2t_ref[...]
    cols = [jnp.sum(jnp.tanh(part_h + part_m[l:l + 1]) * w2, axis=1,
                    keepdims=True) for l in range(nlev)]
    scores = jnp.concatenate(cols, axis=1) + b2_ref[0, 0]
    valid = cntrow_ref[...] > 0.0
    scores = jnp.where(valid, scores, _NEG_INF)
    smax = jnp.max(scores, axis=1, keepdims=True)
    e = jnp.where(valid, jnp.exp(scores - smax), 0.0)
    wts = e / jnp.sum(e, axis=1, keepdims=True)

    enhanced = jax.lax.dot_general(wts, means_h, (((1,), (0,)), ((), ())),
                                   preferred_element_type=jnp.float32)

    out = jax.lax.dot_general(enhanced, Wo_ref[...], (((1,), (0,)), ((), ())),
                              preferred_element_type=jnp.float32)
    out = out + bo_ref[...]
    mu = jnp.mean(out, axis=1, keepdims=True)
    var = jnp.mean((out - mu) * (out - mu), axis=1, keepdims=True)
    out = (out - mu) * jax.lax.rsqrt(var + 1e-5)
    out = out * gamma_ref[...] + beta_ref[...]
    out_ref[...] = jnp.maximum(out, 0.0)


def kernel(node_features, node_levels, W_proj, b_proj, W1, b1, W2, b2,
           Wo, bo, gamma, beta):
    N, IN = node_features.shape
    L, _, HID = W_proj.shape
    OUT = Wo.shape[1]
    B = _B
    NB = N // B
    assert NB * B == N

    lev3 = node_levels.reshape(NB, B, 1).astype(jnp.int32)
    W1a = W1[:HID]
    W1b = W1[HID:]

    full = lambda shape: pl.BlockSpec(shape, lambda i, _s=len(shape): (0,) * _s)

    sums, cnt_row, cnt_b, Wc, bc = pl.pallas_call(
        functools.partial(_seg_kernel, nlev=L),
        grid=(NB,),
        in_specs=[
            pl.BlockSpec((1, B, 1), lambda i: (i, 0, 0)),
            pl.BlockSpec((B, IN), lambda i: (i, 0)),
            full((L, IN, HID)),
            full((HID, HID)),
            full((L, HID)),
        ],
        out_specs=[
            full((L, IN)),
            full((1, L)),
            full((L, IN)),
            full((L, IN, HID)),
            full((L, HID)),
        ],
        out_shape=[
            jax.ShapeDtypeStruct((L, IN), jnp.float32),
            jax.ShapeDtypeStruct((1, L), jnp.float32),
            jax.ShapeDtypeStruct((L, IN), jnp.float32),
            jax.ShapeDtypeStruct((L, IN, HID), jnp.float32),
            jax.ShapeDtypeStruct((L, HID), jnp.float32),
        ],
        compiler_params=pltpu.CompilerParams(
            dimension_semantics=("arbitrary",)),
    )(lev3, node_features, W_proj, W1a, b_proj)

    out = pl.pallas_call(
        functools.partial(_attn_kernel, nlev=L),
        grid=(NB,),
        in_specs=[
            pl.BlockSpec((1, B, 1), lambda i: (i, 0, 0)),
            pl.BlockSpec((B, IN), lambda i: (i, 0)),
            full((L, IN)),
            full((1, L)),
            full((L, IN)),
            full((L, IN, HID)),
            full((L, HID)),
            full((L, IN, HID)),
            full((L, HID)),
            full((HID, HID)),
            full((1, HID)),
            full((1, HID)),
            full((1, 1)),
            full((HID, OUT)),
            full((1, OUT)),
            full((1, OUT)),
            full((1, OUT)),
        ],
        out_specs=pl.BlockSpec((B, OUT), lambda i: (i, 0)),
        out_shape=jax.ShapeDtypeStruct((N, OUT), jnp.float32),
        scratch_shapes=[pltpu.VMEM((B, HID), jnp.float32)],
        compiler_params=pltpu.CompilerParams(
            dimension_semantics=("parallel",)),
    )(lev3, node_features, sums, cnt_row, cnt_b, Wc, bc, W_proj, b_proj,
      W1b, b1.reshape(1, HID), W2.reshape(1, HID), b2.reshape(1, 1),
      Wo, bo.reshape(1, OUT), gamma.reshape(1, OUT), beta.reshape(1, OUT))

    return out


# speedup vs baseline: 6.7560x; 6.7560x over previous
import functools

import jax
import jax.numpy as jnp
from jax.experimental import pallas as pl
from jax.experimental.pallas import tpu as pltpu

_B = 1000
_NEG_INF = float("-inf")


def _seg_kernel(lev_ref, x_ref, Wproj_ref, W1a_ref, bproj_ref,
                sums_ref, cntrow_ref, cntb_ref, Wc_ref, bc_ref, *, nlev):
    i = pl.program_id(0)
    lev = lev_ref[0]
    B = lev.shape[0]
    x = x_ref[...]
    onehot = (lev == jax.lax.broadcasted_iota(jnp.int32, (B, nlev), 1)
              ).astype(jnp.float32)
    psums = jax.lax.dot_general(onehot, x, (((0,), (0,)), ((), ())),
                                preferred_element_type=jnp.float32)
    pcnt_row = jnp.sum(onehot, axis=0, keepdims=True)
    pcnt_b = jax.lax.dot_general(onehot, jnp.ones_like(x),
                                 (((0,), (0,)), ((), ())),
                                 preferred_element_type=jnp.float32)

    @pl.when(i == 0)
    def _():
        sums_ref[...] = psums
        cntrow_ref[...] = pcnt_row
        cntb_ref[...] = pcnt_b
        for l in range(nlev):
            Wc_ref[l] = jax.lax.dot_general(
                Wproj_ref[l], W1a_ref[...], (((1,), (0,)), ((), ())),
                preferred_element_type=jnp.float32)
        bc_ref[...] = jax.lax.dot_general(
            bproj_ref[...], W1a_ref[...], (((1,), (0,)), ((), ())),
            preferred_element_type=jnp.float32)

    @pl.when(i > 0)
    def _():
        sums_ref[...] += psums
        cntrow_ref[...] += pcnt_row
        cntb_ref[...] += pcnt_b


def _attn_kernel(lev_ref, x_ref, sums_ref, cntrow_ref, cntb_ref, Wc_ref,
                 bc_ref, Wproj_ref, bproj_ref, W1b_ref, b1_ref, W2t_ref,
                 b2_ref, Wo_ref, bo_ref, gamma_ref, beta_ref,
                 out_ref, ph_ref, *, nlev):
    lev = lev_ref[0]
    B = lev.shape[0]
    x = x_ref[...]

    cnt_b = cntb_ref[...]
    means_x = sums_ref[...] / jnp.maximum(cnt_b, 1.0)
    rows = [jax.lax.dot_general(means_x[l:l + 1], Wproj_ref[l],
                                (((1,), (0,)), ((), ())),
                                preferred_element_type=jnp.float32)
            for l in range(nlev)]
    means_h = jnp.concatenate(rows, axis=0) + bproj_ref[...]
    means_h = jnp.where(cnt_b > 0.0, means_h, 0.0)
    part_m = jax.lax.dot_general(means_h, W1b_ref[...],
                                 (((1,), (0,)), ((), ())),
                                 preferred_element_type=jnp.float32)
    part_m = part_m + b1_ref[...]

    ph_ref[...] = jnp.zeros_like(ph_ref)
    for l in range(nlev):
        m = lev == l

        @pl.when(jnp.any(m))
        def _(l=l, m=m):
            ph_ref[...] += jax.lax.dot_general(
                jnp.where(m, x, 0.0), Wc_ref[l], (((1,), (0,)), ((), ())),
                preferred_element_type=jnp.float32)

    onehot = (lev == jax.lax.broadcasted_iota(jnp.int32, (B, nlev), 1)
              ).astype(jnp.float32)
    part_h = ph_ref[...] + jax.lax.dot_general(
        onehot, bc_ref[...], (((1,), (0,)), ((), ())),
        preferred_element_type=jnp.float32)

    w2 = W2t_ref[...]
    cols = [jnp.sum(jnp.tanh(part_h + part_m[l:l + 1]) * w2, axis=1,
                    keepdims=True) for l in range(nlev)]
    scores = jnp.concatenate(cols, axis=1) + b2_ref[0, 0]
    valid = cntrow_ref[...] > 0.0
    scores = jnp.where(valid, scores, _NEG_INF)
    smax = jnp.max(scores, axis=1, keepdims=True)
    e = jnp.where(valid, jnp.exp(scores - smax), 0.0)
    wts = e / jnp.sum(e, axis=1, keepdims=True)

    enhanced = jax.lax.dot_general(wts, means_h, (((1,), (0,)), ((), ())),
                                   preferred_element_type=jnp.float32)

    out = jax.lax.dot_general(enhanced, Wo_ref[...], (((1,), (0,)), ((), ())),
                              preferred_element_type=jnp.float32)
    out = out + bo_ref[...]
    mu = jnp.mean(out, axis=1, keepdims=True)
    var = jnp.mean((out - mu) * (out - mu), axis=1, keepdims=True)
    out = (out - mu) * jax.lax.rsqrt(var + 1e-5)
    out = out * gamma_ref[...] + beta_ref[...]
    out_ref[...] = jnp.maximum(out, 0.0)


def kernel(node_features, node_levels, W_proj, b_proj, W1, b1, W2, b2,
           Wo, bo, gamma, beta):
    N, IN = node_features.shape
    L, _, HID = W_proj.shape
    OUT = Wo.shape[1]
    B = _B
    NB = N // B
    assert NB * B == N

    lev3 = node_levels.reshape(NB, B, 1).astype(jnp.int32)
    W1a = W1[:HID]
    W1b = W1[HID:]

    full = lambda shape: pl.BlockSpec(shape, lambda i, _s=len(shape): (0,) * _s)

    sums, cnt_row, cnt_b, Wc, bc = pl.pallas_call(
        functools.partial(_seg_kernel, nlev=L),
        grid=(NB,),
        in_specs=[
            pl.BlockSpec((1, B, 1), lambda i: (i, 0, 0)),
            pl.BlockSpec((B, IN), lambda i: (i, 0)),
            full((L, IN, HID)),
            full((HID, HID)),
            full((L, HID)),
        ],
        out_specs=[
            full((L, IN)),
            full((1, L)),
            full((L, IN)),
            full((L, IN, HID)),
            full((L, HID)),
        ],
        out_shape=[
            jax.ShapeDtypeStruct((L, IN), jnp.float32),
            jax.ShapeDtypeStruct((1, L), jnp.float32),
            jax.ShapeDtypeStruct((L, IN), jnp.float32),
            jax.ShapeDtypeStruct((L, IN, HID), jnp.float32),
            jax.ShapeDtypeStruct((L, HID), jnp.float32),
        ],
        compiler_params=pltpu.CompilerParams(
            dimension_semantics=("arbitrary",)),
    )(lev3, node_features, W_proj, W1a, b_proj)

    out = pl.pallas_call(
        functools.partial(_attn_kernel, nlev=L),
        grid=(NB,),
        in_specs=[
            pl.BlockSpec((1, B, 1), lambda i: (i, 0, 0)),
            pl.BlockSpec((B, IN), lambda i: (i, 0)),
            full((L, IN)),
            full((1, L)),
            full((L, IN)),
            full((L, IN, HID)),
            full((L, HID)),
            full((L, IN, HID)),
            full((L, HID)),
            full((HID, HID)),
            full((1, HID)),
            full((1, HID)),
            full((1, 1)),
            full((HID, OUT)),
            full((1, OUT)),
            full((1, OUT)),
            full((1, OUT)),
        ],
        out_specs=pl.BlockSpec((B, OUT), lambda i: (i, 0)),
        out_shape=jax.ShapeDtypeStruct((N, OUT), jnp.float32),
        scratch_shapes=[pltpu.VMEM((B, HID), jnp.float32)],
        compiler_params=pltpu.CompilerParams(
            dimension_semantics=("parallel",)),
    )(lev3, node_features, sums, cnt_row, cnt_b, Wc, bc, W_proj, b_proj,
      W1b, b1.reshape(1, HID), W2.reshape(1, HID), b2.reshape(1, 1),
      Wo, bo.reshape(1, OUT), gamma.reshape(1, OUT), beta.reshape(1, OUT))

    return out


# speedup vs baseline: 7.0828x; 1.0484x over previous
import functools

import jax
import jax.numpy as jnp
from jax.experimental import pallas as pl
from jax.experimental.pallas import tpu as pltpu

_B = 1000
_NEG_INF = float("-inf")


def _seg_kernel(lev_ref, x_ref, Wproj_ref, W1a_ref, W1b_ref, bproj_ref,
                b1_ref, meansh_ref, partm_ref, cntrow_ref, Wc_ref, bc_ref,
                sums_s, cntb_s, *, nlev, nblocks):
    i = pl.program_id(0)
    lev = lev_ref[0]
    B = lev.shape[0]
    x = x_ref[...]
    onehot = (lev == jax.lax.broadcasted_iota(jnp.int32, (B, nlev), 1)
              ).astype(jnp.float32)
    psums = jax.lax.dot_general(onehot, x, (((0,), (0,)), ((), ())),
                                preferred_element_type=jnp.float32)
    pcnt_b = jax.lax.dot_general(onehot, jnp.ones_like(x),
                                 (((0,), (0,)), ((), ())),
                                 preferred_element_type=jnp.float32)
    pcnt_row = jnp.sum(onehot, axis=0, keepdims=True)

    @pl.when(i == 0)
    def _():
        sums_s[...] = psums
        cntb_s[...] = pcnt_b
        cntrow_ref[...] = pcnt_row
        for l in range(nlev):
            Wc_ref[l] = jax.lax.dot_general(
                Wproj_ref[l], W1a_ref[...], (((1,), (0,)), ((), ())),
                preferred_element_type=jnp.float32)
        bc_ref[...] = jax.lax.dot_general(
            bproj_ref[...], W1a_ref[...], (((1,), (0,)), ((), ())),
            preferred_element_type=jnp.float32)

    @pl.when(i > 0)
    def _():
        sums_s[...] += psums
        cntb_s[...] += pcnt_b
        cntrow_ref[...] += pcnt_row

    @pl.when(i == nblocks - 1)
    def _():
        cnt_b = cntb_s[...]
        means_x = sums_s[...] / jnp.maximum(cnt_b, 1.0)
        rows = [jax.lax.dot_general(means_x[l:l + 1], Wproj_ref[l],
                                    (((1,), (0,)), ((), ())),
                                    preferred_element_type=jnp.float32)
                for l in range(nlev)]
        means_h = jnp.concatenate(rows, axis=0) + bproj_ref[...]
        means_h = jnp.where(cnt_b > 0.0, means_h, 0.0)
        meansh_ref[...] = means_h
        partm_ref[...] = jax.lax.dot_general(
            means_h, W1b_ref[...], (((1,), (0,)), ((), ())),
            preferred_element_type=jnp.float32) + b1_ref[...]


def _attn_kernel(lev_ref, x_ref, meansh_ref, partm_ref, cntrow_ref, Wc_ref,
                 bc_ref, W2_ref, b2_ref, Wo_ref, bo_ref, gamma_ref, beta_ref,
                 out_ref, ph_ref, *, nlev):
    lev = lev_ref[0]
    B = lev.shape[0]
    x = x_ref[...]
    l0 = lev_ref[0, 0, 0]
    l1 = lev_ref[0, B - 1, 0]

    @pl.when(l0 == l1)
    def _():
        W = Wc_ref[pl.ds(l0, 1)].reshape(Wc_ref.shape[1], Wc_ref.shape[2])
        ph_ref[...] = jax.lax.dot_general(
            x, W, (((1,), (0,)), ((), ())),
            preferred_element_type=jnp.float32) + bc_ref[pl.ds(l0, 1)]

    @pl.when(l0 != l1)
    def _():
        acc = jnp.zeros_like(ph_ref)
        onehot = (lev == jax.lax.broadcasted_iota(jnp.int32, (B, nlev), 1)
                  ).astype(jnp.float32)
        acc += jax.lax.dot_general(onehot, bc_ref[...],
                                   (((1,), (0,)), ((), ())),
                                   preferred_element_type=jnp.float32)
        ph_ref[...] = acc
        for l in range(nlev):
            @pl.when((l0 <= l) & (l <= l1))
            def _(l=l):
                m = lev == l
                ph_ref[...] += jax.lax.dot_general(
                    jnp.where(m, x, 0.0), Wc_ref[l],
                    (((1,), (0,)), ((), ())),
                    preferred_element_type=jnp.float32)

    ph2 = 2.0 * ph_ref[...]
    pm2 = 2.0 * partm_ref[...]

    cols = []
    for l in range(nlev):
        t = 1.0 - 2.0 / (1.0 + jnp.exp(ph2 + pm2[l:l + 1]))
        cols.append(jax.lax.dot_general(t, W2_ref[...],
                                        (((1,), (0,)), ((), ())),
                                        preferred_element_type=jnp.float32))
    scores = jnp.concatenate(cols, axis=1) + b2_ref[0, 0]
    valid = cntrow_ref[...] > 0.0
    scores = jnp.where(valid, scores, _NEG_INF)
    smax = jnp.max(scores, axis=1, keepdims=True)
    e = jnp.where(valid, jnp.exp(scores - smax), 0.0)
    wts = e / jnp.sum(e, axis=1, keepdims=True)

    enhanced = jax.lax.dot_general(wts, meansh_ref[...],
                                   (((1,), (0,)), ((), ())),
                                   preferred_element_type=jnp.float32)

    out = jax.lax.dot_general(enhanced, Wo_ref[...], (((1,), (0,)), ((), ())),
                              preferred_element_type=jnp.float32)
    out = out + bo_ref[...]
    mu = jnp.mean(out, axis=1, keepdims=True)
    var = jnp.mean((out - mu) * (out - mu), axis=1, keepdims=True)
    out = (out - mu) * jax.lax.rsqrt(var + 1e-5)
    out = out * gamma_ref[...] + beta_ref[...]
    out_ref[...] = jnp.maximum(out, 0.0)


def kernel(node_features, node_levels, W_proj, b_proj, W1, b1, W2, b2,
           Wo, bo, gamma, beta):
    N, IN = node_features.shape
    L, _, HID = W_proj.shape
    OUT = Wo.shape[1]
    B = _B
    NB = N // B
    assert NB * B == N

    lev3 = node_levels.reshape(NB, B, 1).astype(jnp.int32)
    W1a = W1[:HID]
    W1b = W1[HID:]

    full = lambda shape: pl.BlockSpec(shape, lambda i, _s=len(shape): (0,) * _s)

    means_h, part_m, cnt_row, Wc, bc = pl.pallas_call(
        functools.partial(_seg_kernel, nlev=L, nblocks=NB),
        grid=(NB,),
        in_specs=[
            pl.BlockSpec((1, B, 1), lambda i: (i, 0, 0)),
            pl.BlockSpec((B, IN), lambda i: (i, 0)),
            full((L, IN, HID)),
            full((HID, HID)),
            full((HID, HID)),
            full((L, HID)),
            full((1, HID)),
        ],
        out_specs=[
            full((L, HID)),
            full((L, HID)),
            full((1, L)),
            full((L, IN, HID)),
            full((L, HID)),
        ],
        out_shape=[
            jax.ShapeDtypeStruct((L, HID), jnp.float32),
            jax.ShapeDtypeStruct((L, HID), jnp.float32),
            jax.ShapeDtypeStruct((1, L), jnp.float32),
            jax.ShapeDtypeStruct((L, IN, HID), jnp.float32),
            jax.ShapeDtypeStruct((L, HID), jnp.float32),
        ],
        scratch_shapes=[pltpu.VMEM((L, IN), jnp.float32),
                        pltpu.VMEM((L, IN), jnp.float32)],
        compiler_params=pltpu.CompilerParams(
            dimension_semantics=("arbitrary",)),
    )(lev3, node_features, W_proj, W1a, W1b, b_proj, b1.reshape(1, HID))

    out = pl.pallas_call(
        functools.partial(_attn_kernel, nlev=L),
        grid=(NB,),
        in_specs=[
            pl.BlockSpec((1, B, 1), lambda i: (i, 0, 0)),
            pl.BlockSpec((B, IN), lambda i: (i, 0)),
            full((L, HID)),
            full((L, HID)),
            full((1, L)),
            full((L, IN, HID)),
            full((L, HID)),
            full((HID, 1)),
            full((1, 1)),
            full((HID, OUT)),
            full((1, OUT)),
            full((1, OUT)),
            full((1, OUT)),
        ],
        out_specs=pl.BlockSpec((B, OUT), lambda i: (i, 0)),
        out_shape=jax.ShapeDtypeStruct((N, OUT), jnp.float32),
        scratch_shapes=[pltpu.VMEM((B, HID), jnp.float32)],
        compiler_params=pltpu.CompilerParams(
            dimension_semantics=("parallel",)),
    )(lev3, node_features, means_h, part_m, cnt_row, Wc, bc,
      W2, b2.reshape(1, 1), Wo, bo.reshape(1, OUT), gamma.reshape(1, OUT),
      beta.reshape(1, OUT))

    return out


# speedup vs baseline: 7.2216x; 1.0196x over previous
import functools

import jax
import jax.numpy as jnp
from jax.experimental import pallas as pl
from jax.experimental.pallas import tpu as pltpu

_B = 1000
_NEG_INF = float("-inf")


def _seg_kernel(lev_ref, x_ref, Wproj_ref, W1a_ref, W1b_ref, bproj_ref,
                b1_ref, W2_ref, b2_ref,
                meansh_ref, fm_ref, cntrow_ref, Wc_ref, bc_ref, sbase_ref,
                w2m2_ref, sums_s, cntb_s, *, nlev, nblocks):
    i = pl.program_id(0)
    lev = lev_ref[0]
    B = lev.shape[0]
    x = x_ref[...]
    l0 = lev_ref[0, 0, 0]
    l1 = lev_ref[0, B - 1, 0]

    @pl.when(i == 0)
    def _():
        sums_s[...] = jnp.zeros_like(sums_s)
        cntb_s[...] = jnp.zeros_like(cntb_s)
        cntrow_ref[...] = jnp.zeros_like(cntrow_ref)
        for l in range(nlev):
            Wc_ref[l] = jax.lax.dot_general(
                Wproj_ref[l], W1a_ref[...], (((1,), (0,)), ((), ())),
                preferred_element_type=jnp.float32)
        bc_ref[...] = jax.lax.dot_general(
            bproj_ref[...], W1a_ref[...], (((1,), (0,)), ((), ())),
            preferred_element_type=jnp.float32)

    @pl.when(l0 == l1)
    def _():
        rowsum = jnp.sum(x, axis=0, keepdims=True)
        sums_s[pl.ds(l0, 1)] += rowsum
        onerow = (jax.lax.broadcasted_iota(jnp.int32, cntb_s.shape, 0) == l0
                  ).astype(jnp.float32)
        cntb_s[...] += onerow * float(B)
        cntrow_ref[...] += (
            jax.lax.broadcasted_iota(jnp.int32, cntrow_ref.shape, 1) == l0
        ).astype(jnp.float32) * float(B)

    @pl.when(l0 != l1)
    def _():
        onehot = (lev == jax.lax.broadcasted_iota(jnp.int32, (B, nlev), 1)
                  ).astype(jnp.float32)
        sums_s[...] += jax.lax.dot_general(
            onehot, x, (((0,), (0,)), ((), ())),
            preferred_element_type=jnp.float32)
        cntb_s[...] += jax.lax.dot_general(
            onehot, jnp.ones_like(x), (((0,), (0,)), ((), ())),
            preferred_element_type=jnp.float32)
        cntrow_ref[...] += jnp.sum(onehot, axis=0, keepdims=True)

    @pl.when(i == nblocks - 1)
    def _():
        cnt_b = cntb_s[...]
        means_x = sums_s[...] / jnp.maximum(cnt_b, 1.0)
        rows = [jax.lax.dot_general(means_x[l:l + 1], Wproj_ref[l],
                                    (((1,), (0,)), ((), ())),
                                    preferred_element_type=jnp.float32)
                for l in range(nlev)]
        means_h = jnp.concatenate(rows, axis=0) + bproj_ref[...]
        means_h = jnp.where(cnt_b > 0.0, means_h, 0.0)
        meansh_ref[...] = means_h
        part_m = jax.lax.dot_general(
            means_h, W1b_ref[...], (((1,), (0,)), ((), ())),
            preferred_element_type=jnp.float32) + b1_ref[...]
        fm_ref[...] = jnp.exp(2.0 * part_m)
        w2 = W2_ref[...]
        sbase_ref[...] = b2_ref[...] + jnp.sum(w2)
        w2m2_ref[...] = -2.0 * w2


def _attn_kernel(lev_ref, x_ref, meansh_ref, fm_ref, cntrow_ref, Wc_ref,
                 bc_ref, sbase_ref, w2m2_ref, Wo_ref, bo_ref, gamma_ref,
                 beta_ref, out_ref, ph_ref, *, nlev):
    lev = lev_ref[0]
    B = lev.shape[0]
    x = x_ref[...]
    l0 = lev_ref[0, 0, 0]
    l1 = lev_ref[0, B - 1, 0]

    @pl.when(l0 == l1)
    def _():
        W = Wc_ref[pl.ds(l0, 1)].reshape(Wc_ref.shape[1], Wc_ref.shape[2])
        ph_ref[...] = jax.lax.dot_general(
            x, W, (((1,), (0,)), ((), ())),
            preferred_element_type=jnp.float32) + bc_ref[pl.ds(l0, 1)]

    @pl.when(l0 != l1)
    def _():
        onehot = (lev == jax.lax.broadcasted_iota(jnp.int32, (B, nlev), 1)
                  ).astype(jnp.float32)
        ph_ref[...] = jax.lax.dot_general(
            onehot, bc_ref[...], (((1,), (0,)), ((), ())),
            preferred_element_type=jnp.float32)
        for l in range(nlev):
            @pl.when((l0 <= l) & (l <= l1))
            def _(l=l):
                m = lev == l
                ph_ref[...] += jax.lax.dot_general(
                    jnp.where(m, x, 0.0), Wc_ref[l],
                    (((1,), (0,)), ((), ())),
                    preferred_element_type=jnp.float32)

    E = jnp.exp(2.0 * ph_ref[...])
    w2m2 = w2m2_ref[...].astype(jnp.bfloat16)
    fm = fm_ref[...]

    cols = []
    for l in range(nlev):
        d = (1.0 / (1.0 + E * fm[l:l + 1])).astype(jnp.bfloat16)
        cols.append(jax.lax.dot_general(d, w2m2, (((1,), (0,)), ((), ())),
                                        preferred_element_type=jnp.float32))
    scores = jnp.concatenate(cols, axis=1) + sbase_ref[0, 0]
    valid = cntrow_ref[...] > 0.0
    scores = jnp.where(valid, scores, _NEG_INF)
    smax = jnp.max(scores, axis=1, keepdims=True)
    e = jnp.where(valid, jnp.exp(scores - smax), 0.0)
    wts = e / jnp.sum(e, axis=1, keepdims=True)

    enhanced = jax.lax.dot_general(wts, meansh_ref[...],
                                   (((1,), (0,)), ((), ())),
                                   preferred_element_type=jnp.float32)

    out = jax.lax.dot_general(enhanced, Wo_ref[...], (((1,), (0,)), ((), ())),
                              preferred_element_type=jnp.float32)
    out = out + bo_ref[...]
    mu = jnp.mean(out, axis=1, keepdims=True)
    var = jnp.mean((out - mu) * (out - mu), axis=1, keepdims=True)
    out = (out - mu) * jax.lax.rsqrt(var + 1e-5)
    out = out * gamma_ref[...] + beta_ref[...]
    out_ref[...] = jnp.maximum(out, 0.0)


def kernel(node_features, node_levels, W_proj, b_proj, W1, b1, W2, b2,
           Wo, bo, gamma, beta):
    N, IN = node_features.shape
    L, _, HID = W_proj.shape
    OUT = Wo.shape[1]
    B = _B
    NB = N // B
    assert NB * B == N

    lev3 = node_levels.reshape(NB, B, 1).astype(jnp.int32)
    W1a = W1[:HID]
    W1b = W1[HID:]

    full = lambda shape: pl.BlockSpec(shape, lambda i, _s=len(shape): (0,) * _s)

    means_h, fm, cnt_row, Wc, bc, sbase, w2m2 = pl.pallas_call(
        functools.partial(_seg_kernel, nlev=L, nblocks=NB),
        grid=(NB,),
        in_specs=[
            pl.BlockSpec((1, B, 1), lambda i: (i, 0, 0)),
            pl.BlockSpec((B, IN), lambda i: (i, 0)),
            full((L, IN, HID)),
            full((HID, HID)),
            full((HID, HID)),
            full((L, HID)),
            full((1, HID)),
            full((HID, 1)),
            full((1, 1)),
        ],
        out_specs=[
            full((L, HID)),
            full((L, HID)),
            full((1, L)),
            full((L, IN, HID)),
            full((L, HID)),
            full((1, 1)),
            full((HID, 1)),
        ],
        out_shape=[
            jax.ShapeDtypeStruct((L, HID), jnp.float32),
            jax.ShapeDtypeStruct((L, HID), jnp.float32),
            jax.ShapeDtypeStruct((1, L), jnp.float32),
            jax.ShapeDtypeStruct((L, IN, HID), jnp.float32),
            jax.ShapeDtypeStruct((L, HID), jnp.float32),
            jax.ShapeDtypeStruct((1, 1), jnp.float32),
            jax.ShapeDtypeStruct((HID, 1), jnp.float32),
        ],
        scratch_shapes=[pltpu.VMEM((L, IN), jnp.float32),
                        pltpu.VMEM((L, IN), jnp.float32)],
        compiler_params=pltpu.CompilerParams(
            dimension_semantics=("arbitrary",)),
    )(lev3, node_features, W_proj, W1a, W1b, b_proj, b1.reshape(1, HID),
      W2, b2.reshape(1, 1))

    out = pl.pallas_call(
        functools.partial(_attn_kernel, nlev=L),
        grid=(NB,),
        in_specs=[
            pl.BlockSpec((1, B, 1), lambda i: (i, 0, 0)),
            pl.BlockSpec((B, IN), lambda i: (i, 0)),
            full((L, HID)),
            full((L, HID)),
            full((1, L)),
            full((L, IN, HID)),
            full((L, HID)),
            full((1, 1)),
            full((HID, 1)),
            full((HID, OUT)),
            full((1, OUT)),
            full((1, OUT)),
            full((1, OUT)),
        ],
        out_specs=pl.BlockSpec((B, OUT), lambda i: (i, 0)),
        out_shape=jax.ShapeDtypeStruct((N, OUT), jnp.float32),
        scratch_shapes=[pltpu.VMEM((B, HID), jnp.float32)],
        compiler_params=pltpu.CompilerParams(
            dimension_semantics=("parallel",)),
    )(lev3, node_features, means_h, fm, cnt_row, Wc, bc, sbase, w2m2,
      Wo, bo.reshape(1, OUT), gamma.reshape(1, OUT), beta.reshape(1, OUT))

    return out


# speedup vs baseline: 8.3497x; 1.1562x over previous
import functools

import jax
import jax.numpy as jnp
from jax.experimental import pallas as pl
from jax.experimental.pallas import tpu as pltpu

_B = 5000
_NEG_INF = float("-inf")


def _seg_kernel(lev_ref, x_ref, Wproj_ref, W1a_ref, W1b_ref, bproj_ref,
                b1_ref, W2_ref, b2_ref,
                meansh_ref, fm_ref, cntrow_ref, Wc_ref, dWc_ref, bc_ref,
                dbc_ref, sbase_ref, w2m2_ref, sums_s, *, nlev, nblocks):
    i = pl.program_id(0)
    lev = lev_ref[0]
    B = lev.shape[0]
    x = x_ref[...]
    l0 = lev_ref[0, 0, 0]
    l1 = lev_ref[0, B - 1, 0]

    @pl.when(i == 0)
    def _():
        sums_s[...] = jnp.zeros_like(sums_s)
        cntrow_ref[...] = jnp.zeros_like(cntrow_ref)
        for l in range(nlev):
            Wc_ref[l] = jax.lax.dot_general(
                Wproj_ref[l], W1a_ref[...], (((1,), (0,)), ((), ())),
                preferred_element_type=jnp.float32)
        dWc_ref[0] = Wc_ref[0]
        for l in range(1, nlev):
            dWc_ref[l] = Wc_ref[l] - Wc_ref[l - 1]
        bc_ref[...] = jax.lax.dot_general(
            bproj_ref[...], W1a_ref[...], (((1,), (0,)), ((), ())),
            preferred_element_type=jnp.float32)
        dbc = bc_ref[...]
        dbc_ref[...] = dbc - jnp.concatenate(
            [jnp.zeros_like(dbc[0:1]), dbc[:-1]], axis=0)

    @pl.when(l0 == l1)
    def _():
        sums_s[pl.ds(l0, 1)] += jnp.sum(x, axis=0, keepdims=True)
        cntrow_ref[...] += (
            jax.lax.broadcasted_iota(jnp.int32, cntrow_ref.shape, 1) == l0
        ).astype(jnp.float32) * float(B)

    @pl.when(l0 != l1)
    def _():
        for l in range(nlev):
            @pl.when((l0 <= l) & (l <= l1))
            def _(l=l):
                m = lev == l
                sums_s[pl.ds(l, 1)] += jnp.sum(
                    jnp.where(m, x, 0.0), axis=0, keepdims=True)
        onehot = (lev == jax.lax.broadcasted_iota(jnp.int32, (B, nlev), 1)
                  ).astype(jnp.float32)
        cntrow_ref[...] += jnp.sum(onehot, axis=0, keepdims=True)

    @pl.when(i == nblocks - 1)
    def _():
        rows = []
        for l in range(nlev):
            c = cntrow_ref[0, l]
            row = jax.lax.dot_general(
                sums_s[l:l + 1] * (1.0 / jnp.maximum(c, 1.0)),
                Wproj_ref[l], (((1,), (0,)), ((), ())),
                preferred_element_type=jnp.float32) + bproj_ref[l:l + 1]
            rows.append(jnp.where(c > 0.0, row, jnp.zeros_like(row)))
        means_h = jnp.concatenate(rows, axis=0)
        meansh_ref[...] = means_h
        part_m = jax.lax.dot_general(
            means_h, W1b_ref[...], (((1,), (0,)), ((), ())),
            preferred_element_type=jnp.float32) + b1_ref[...]
        fm_ref[...] = jnp.exp(2.0 * part_m)
        w2 = W2_ref[...]
        sbase_ref[...] = b2_ref[...] + jnp.sum(w2)
        w2m2_ref[...] = -2.0 * w2


def _attn_kernel(lev_ref, x_ref, meansh_ref, fm_ref, cntrow_ref, Wc_ref,
                 dWc_ref, bc_ref, dbc_ref, sbase_ref, w2m2_ref, Wo_ref,
                 bo_ref, gamma_ref, beta_ref, out_ref, ph_ref, *, nlev):
    lev = lev_ref[0]
    B = lev.shape[0]
    x = x_ref[...]
    l0 = lev_ref[0, 0, 0]
    l1 = lev_ref[0, B - 1, 0]

    W = Wc_ref[pl.ds(l0, 1)].reshape(Wc_ref.shape[1], Wc_ref.shape[2])
    ph_ref[...] = jax.lax.dot_general(
        x, W, (((1,), (0,)), ((), ())),
        preferred_element_type=jnp.float32) + bc_ref[pl.ds(l0, 1)]
    for l in range(1, nlev):
        @pl.when((l0 < l) & (l <= l1))
        def _(l=l):
            m = lev >= l
            ph_ref[...] += jax.lax.dot_general(
                jnp.where(m, x, 0.0), dWc_ref[l], (((1,), (0,)), ((), ())),
                preferred_element_type=jnp.float32
            ) + jnp.where(m, dbc_ref[l:l + 1], 0.0)

    E = jnp.exp(2.0 * ph_ref[...])
    w2m2 = w2m2_ref[...].astype(jnp.bfloat16)
    fm = fm_ref[...]

    cols = []
    for l in range(nlev):
        d = (1.0 / (1.0 + E * fm[l:l + 1])).astype(jnp.bfloat16)
        cols.append(jax.lax.dot_general(d, w2m2, (((1,), (0,)), ((), ())),
                                        preferred_element_type=jnp.float32))
    scores = jnp.concatenate(cols, axis=1) + sbase_ref[0, 0]
    valid = cntrow_ref[...] > 0.0
    scores = jnp.where(valid, scores, _NEG_INF)
    smax = jnp.max(scores, axis=1, keepdims=True)
    e = jnp.where(valid, jnp.exp(scores - smax), 0.0)
    wts = e / jnp.sum(e, axis=1, keepdims=True)

    enhanced = jax.lax.dot_general(wts, meansh_ref[...],
                                   (((1,), (0,)), ((), ())),
                                   preferred_element_type=jnp.float32)

    out = jax.lax.dot_general(enhanced, Wo_ref[...], (((1,), (0,)), ((), ())),
                              preferred_element_type=jnp.float32)
    out = out + bo_ref[...]
    mu = jnp.mean(out, axis=1, keepdims=True)
    var = jnp.mean((out - mu) * (out - mu), axis=1, keepdims=True)
    out = (out - mu) * jax.lax.rsqrt(var + 1e-5)
    out = out * gamma_ref[...] + beta_ref[...]
    out_ref[...] = jnp.maximum(out, 0.0)


def kernel(node_features, node_levels, W_proj, b_proj, W1, b1, W2, b2,
           Wo, bo, gamma, beta):
    N, IN = node_features.shape
    L, _, HID = W_proj.shape
    OUT = Wo.shape[1]
    B = _B
    NB = N // B
    assert NB * B == N

    lev3 = node_levels.reshape(NB, B, 1).astype(jnp.int32)
    W1a = W1[:HID]
    W1b = W1[HID:]

    full = lambda shape: pl.BlockSpec(shape, lambda i, _s=len(shape): (0,) * _s)

    (means_h, fm, cnt_row, Wc, dWc, bc, dbc, sbase, w2m2) = pl.pallas_call(
        functools.partial(_seg_kernel, nlev=L, nblocks=NB),
        grid=(NB,),
        in_specs=[
            pl.BlockSpec((1, B, 1), lambda i: (i, 0, 0)),
            pl.BlockSpec((B, IN), lambda i: (i, 0)),
            full((L, IN, HID)),
            full((HID, HID)),
            full((HID, HID)),
            full((L, HID)),
            full((1, HID)),
            full((HID, 1)),
            full((1, 1)),
        ],
        out_specs=[
            full((L, HID)),
            full((L, HID)),
            full((1, L)),
            full((L, IN, HID)),
            full((L, IN, HID)),
            full((L, HID)),
            full((L, HID)),
            full((1, 1)),
            full((HID, 1)),
        ],
        out_shape=[
            jax.ShapeDtypeStruct((L, HID), jnp.float32),
            jax.ShapeDtypeStruct((L, HID), jnp.float32),
            jax.ShapeDtypeStruct((1, L), jnp.float32),
            jax.ShapeDtypeStruct((L, IN, HID), jnp.float32),
            jax.ShapeDtypeStruct((L, IN, HID), jnp.float32),
            jax.ShapeDtypeStruct((L, HID), jnp.float32),
            jax.ShapeDtypeStruct((L, HID), jnp.float32),
            jax.ShapeDtypeStruct((1, 1), jnp.float32),
            jax.ShapeDtypeStruct((HID, 1), jnp.float32),
        ],
        scratch_shapes=[pltpu.VMEM((L, IN), jnp.float32)],
        compiler_params=pltpu.CompilerParams(
            dimension_semantics=("arbitrary",)),
    )(lev3, node_features, W_proj, W1a, W1b, b_proj, b1.reshape(1, HID),
      W2, b2.reshape(1, 1))

    out = pl.pallas_call(
        functools.partial(_attn_kernel, nlev=L),
        grid=(NB,),
        in_specs=[
            pl.BlockSpec((1, B, 1), lambda i: (i, 0, 0)),
            pl.BlockSpec((B, IN), lambda i: (i, 0)),
            full((L, HID)),
            full((L, HID)),
            full((1, L)),
            full((L, IN, HID)),
            full((L, IN, HID)),
            full((L, HID)),
            full((L, HID)),
            full((1, 1)),
            full((HID, 1)),
            full((HID, OUT)),
            full((1, OUT)),
            full((1, OUT)),
            full((1, OUT)),
        ],
        out_specs=pl.BlockSpec((B, OUT), lambda i: (i, 0)),
        out_shape=jax.ShapeDtypeStruct((N, OUT), jnp.float32),
        scratch_shapes=[pltpu.VMEM((B, HID), jnp.float32)],
        compiler_params=pltpu.CompilerParams(
            dimension_semantics=("parallel",)),
    )(lev3, node_features, means_h, fm, cnt_row, Wc, dWc, bc, dbc, sbase,
      w2m2, Wo, bo.reshape(1, OUT), gamma.reshape(1, OUT),
      beta.reshape(1, OUT))

    return out


# speedup vs baseline: 16.4080x; 1.9651x over previous
import functools

import jax
import jax.numpy as jnp
from jax.experimental import pallas as pl
from jax.experimental.pallas import tpu as pltpu

_B = 5000
_NEG_INF = float("-inf")


def _seg_kernel(lev_ref, x_ref, Wproj_ref, W1a_ref, W1b_ref, bproj_ref,
                b1_ref, W2_ref, Wo_ref, bo_ref,
                pmb_ref, cntrow_ref, Wcb_ref, dWcb_ref, bc_ref, dbc_ref,
                w2bd_ref, mo_ref, sums_s, Wc_s, *, nlev, nblocks):
    i = pl.program_id(0)
    lev = lev_ref[0]
    B = lev.shape[0]
    hid = W1a_ref.shape[1]

    @pl.when(i == 0)
    def _():
        sums_s[...] = jnp.zeros_like(sums_s)
        cntrow_ref[...] = jnp.zeros_like(cntrow_ref)

    onehot = (lev == jax.lax.broadcasted_iota(jnp.int32, (B, nlev), 1))
    oh_bf = onehot.astype(jnp.bfloat16)
    x_bf = x_ref[...].astype(jnp.bfloat16)
    sums_s[...] += jax.lax.dot_general(
        oh_bf, x_bf, (((0,), (0,)), ((), ())),
        preferred_element_type=jnp.float32)
    cntrow_ref[...] += jnp.sum(onehot.astype(jnp.float32), axis=0,
                               keepdims=True)

    @pl.when(i == nblocks - 1)
    def _():
        for l in range(nlev):
            Wc_s[l] = jax.lax.dot_general(
                Wproj_ref[l], W1a_ref[...], (((1,), (0,)), ((), ())),
                preferred_element_type=jnp.float32)
        Wcb_ref[...] = Wc_s[...].astype(jnp.bfloat16)
        dWcb_ref[0] = Wcb_ref[0]
        for l in range(1, nlev):
            dWcb_ref[l] = (Wc_s[l] - Wc_s[l - 1]).astype(jnp.bfloat16)
        bc = jax.lax.dot_general(
            bproj_ref[...], W1a_ref[...], (((1,), (0,)), ((), ())),
            preferred_element_type=jnp.float32)
        bc_ref[...] = bc
        dbc_ref[...] = bc - jnp.concatenate(
            [jnp.zeros_like(bc[0:1]), bc[:-1]], axis=0)

        rows = []
        for l in range(nlev):
            c = cntrow_ref[0, l]
            row = jax.lax.dot_general(
                sums_s[l:l + 1] * (1.0 / jnp.maximum(c, 1.0)),
                Wproj_ref[l], (((1,), (0,)), ((), ())),
                preferred_element_type=jnp.float32) + bproj_ref[l:l + 1]
            rows.append(jnp.where(c > 0.0, row, jnp.zeros_like(row)))
        means_h = jnp.concatenate(rows, axis=0)
        part_m = jax.lax.dot_general(
            means_h, W1b_ref[...], (((1,), (0,)), ((), ())),
            preferred_element_type=jnp.float32) + b1_ref[...]
        pmb_ref[...] = part_m.astype(jnp.bfloat16)
        mo_ref[...] = jax.lax.dot_general(
            means_h, Wo_ref[...], (((1,), (0,)), ((), ())),
            preferred_element_type=jnp.float32) + bo_ref[...]

        w2rep = jnp.concatenate([W2_ref[...]] * nlev, axis=0)
        blk = jax.lax.broadcasted_iota(jnp.int32, (nlev * hid, nlev), 0) // hid
        col = jax.lax.broadcasted_iota(jnp.int32, (nlev * hid, nlev), 1)
        w2bd_ref[...] = jnp.where(blk == col, w2rep, 0.0).astype(jnp.bfloat16)


def _attn_kernel(lev_ref, x_ref, pmb_ref, cntrow_ref, Wcb_ref, dWcb_ref,
                 bc_ref, dbc_ref, w2bd_ref, b2_ref, mo_ref, gamma_ref,
                 beta_ref, out_ref, ph_ref, *, nlev):
    lev = lev_ref[0]
    B = lev.shape[0]
    x_bf = x_ref[...].astype(jnp.bfloat16)
    l0 = lev_ref[0, 0, 0]
    l1 = lev_ref[0, B - 1, 0]

    W = Wcb_ref[pl.ds(l0, 1)].reshape(Wcb_ref.shape[1], Wcb_ref.shape[2])
    ph_ref[...] = jax.lax.dot_general(
        x_bf, W, (((1,), (0,)), ((), ())),
        preferred_element_type=jnp.float32) + bc_ref[pl.ds(l0, 1)]
    for l in range(1, nlev):
        @pl.when((l0 < l) & (l <= l1))
        def _(l=l):
            m = lev >= l
            ph_ref[...] += jax.lax.dot_general(
                jnp.where(m, x_bf, 0.0), dWcb_ref[l], (((1,), (0,)), ((), ())),
                preferred_element_type=jnp.float32
            ) + jnp.where(m, dbc_ref[l:l + 1], 0.0)

    ph_bf = ph_ref[...].astype(jnp.bfloat16)
    pmb = pmb_ref[...]

    dall = jnp.concatenate(
        [jnp.tanh(ph_bf + pmb[l:l + 1]) for l in range(nlev)],
        axis=1)
    scores = jax.lax.dot_general(
        dall, w2bd_ref[...], (((1,), (0,)), ((), ())),
        preferred_element_type=jnp.float32) + b2_ref[0, 0]
    valid = cntrow_ref[...] > 0.0
    scores = jnp.where(valid, scores, _NEG_INF)
    smax = jnp.max(scores, axis=1, keepdims=True)
    e = jnp.where(valid, jnp.exp(scores - smax), 0.0)
    wts = e / jnp.sum(e, axis=1, keepdims=True)

    out = jax.lax.dot_general(wts, mo_ref[...], (((1,), (0,)), ((), ())),
                              preferred_element_type=jnp.float32)
    mu = jnp.mean(out, axis=1, keepdims=True)
    var = jnp.mean((out - mu) * (out - mu), axis=1, keepdims=True)
    out = (out - mu) * jax.lax.rsqrt(var + 1e-5)
    out = out * gamma_ref[...] + beta_ref[...]
    out_ref[...] = jnp.maximum(out, 0.0)


def kernel(node_features, node_levels, W_proj, b_proj, W1, b1, W2, b2,
           Wo, bo, gamma, beta):
    N, IN = node_features.shape
    L, _, HID = W_proj.shape
    OUT = Wo.shape[1]
    B = _B
    NB = N // B
    assert NB * B == N

    lev3 = node_levels.reshape(NB, B, 1).astype(jnp.int32)
    W1a = W1[:HID]
    W1b = W1[HID:]

    full = lambda shape: pl.BlockSpec(shape, lambda i, _s=len(shape): (0,) * _s)

    (pmb, cnt_row, Wcb, dWcb, bc, dbc, w2bd, mo) = pl.pallas_call(
        functools.partial(_seg_kernel, nlev=L, nblocks=NB),
        grid=(NB,),
        in_specs=[
            pl.BlockSpec((1, B, 1), lambda i: (i, 0, 0)),
            pl.BlockSpec((B, IN), lambda i: (i, 0)),
            full((L, IN, HID)),
            full((HID, HID)),
            full((HID, HID)),
            full((L, HID)),
            full((1, HID)),
            full((HID, 1)),
            full((HID, OUT)),
            full((1, OUT)),
        ],
        out_specs=[
            full((L, HID)),
            full((1, L)),
            full((L, IN, HID)),
            full((L, IN, HID)),
            full((L, HID)),
            full((L, HID)),
            full((L * HID, L)),
            full((L, OUT)),
        ],
        out_shape=[
            jax.ShapeDtypeStruct((L, HID), jnp.bfloat16),
            jax.ShapeDtypeStruct((1, L), jnp.float32),
            jax.ShapeDtypeStruct((L, IN, HID), jnp.bfloat16),
            jax.ShapeDtypeStruct((L, IN, HID), jnp.bfloat16),
            jax.ShapeDtypeStruct((L, HID), jnp.float32),
            jax.ShapeDtypeStruct((L, HID), jnp.float32),
            jax.ShapeDtypeStruct((L * HID, L), jnp.bfloat16),
            jax.ShapeDtypeStruct((L, OUT), jnp.float32),
        ],
        scratch_shapes=[pltpu.VMEM((L, IN), jnp.float32),
                        pltpu.VMEM((L, IN, HID), jnp.float32)],
        compiler_params=pltpu.CompilerParams(
            dimension_semantics=("arbitrary",)),
    )(lev3, node_features, W_proj, W1a, W1b, b_proj, b1.reshape(1, HID),
      W2, Wo, bo.reshape(1, OUT))

    out = pl.pallas_call(
        functools.partial(_attn_kernel, nlev=L),
        grid=(NB,),
        in_specs=[
            pl.BlockSpec((1, B, 1), lambda i: (i, 0, 0)),
            pl.BlockSpec((B, IN), lambda i: (i, 0)),
            full((L, HID)),
            full((1, L)),
            full((L, IN, HID)),
            full((L, IN, HID)),
            full((L, HID)),
            full((L, HID)),
            full((L * HID, L)),
            full((1, 1)),
            full((L, OUT)),
            full((1, OUT)),
            full((1, OUT)),
        ],
        out_specs=pl.BlockSpec((B, OUT), lambda i: (i, 0)),
        out_shape=jax.ShapeDtypeStruct((N, OUT), jnp.float32),
        scratch_shapes=[pltpu.VMEM((B, HID), jnp.float32)],
        compiler_params=pltpu.CompilerParams(
            dimension_semantics=("parallel",)),
    )(lev3, node_features, pmb, cnt_row, Wcb, dWcb, bc, dbc, w2bd,
      b2.reshape(1, 1), mo, gamma.reshape(1, OUT), beta.reshape(1, OUT))

    return out
